# baseline (speedup 1.0000x reference)
import functools

import jax
import jax.numpy as jnp
from jax import lax
from jax.experimental import pallas as pl
from jax.experimental.pallas import tpu as pltpu

_LANE = 128


def _extrema_kernel(win_hbm, o_hbm, winbuf, bufs, sems, rsem, *,
                    pool_size: int, width: int, region: int, zrows: int,
                    n_chunks: int, n_bufs: int):
    row = o_hbm.shape[1]
    pp = winbuf.shape[2]

    pltpu.make_async_copy(win_hbm, winbuf, rsem).start()
    bufs[...] = jnp.zeros(bufs.shape, bufs.dtype)
    pltpu.make_async_copy(win_hbm, winbuf, rsem).wait()

    dcol = lax.broadcasted_iota(jnp.int32, (1, region), 1)

    for k in range(n_chunks):
        slot = k % n_bufs
        if k >= n_bufs:
            pltpu.make_async_copy(
                bufs.at[slot], o_hbm.at[pl.ds((k - n_bufs) * zrows, zrows), :],
                sems.at[slot]).wait()
        win = winbuf[pl.ds(k * zrows, zrows)]
        awin = jnp.abs(win)
        jidx = jnp.argmax(awin, axis=-1, keepdims=True).astype(jnp.int32)
        wcol2 = lax.broadcasted_iota(jnp.int32, (1, pp), 1)
        hit = wcol2 == jidx[:, 0, :]
        for c in range(1, winbuf.shape[1]):
            hit = hit | (wcol2 == jidx[:, c, :])
        strip = jnp.where(hit, win[:, 0, :], 0.0)
        acc = jnp.zeros((zrows, region), bufs.dtype)
        for j in range(pp):
            acc = jnp.where(dcol == (j // pool_size) * width + j % pool_size,
                            strip[:, j:j + 1], acc)
        bufs[slot, :, :region] = acc
        pltpu.make_async_copy(
            bufs.at[slot], o_hbm.at[pl.ds(k * zrows, zrows), :],
            sems.at[slot]).start()

    for k in range(max(0, n_chunks - n_bufs), n_chunks):
        pltpu.make_async_copy(
            bufs.at[k % n_bufs], o_hbm.at[pl.ds(k * zrows, zrows), :],
            sems.at[k % n_bufs]).wait()


def _extrema_pool_indices_2d(x, pool_size: int):
    N, C, H, W = x.shape
    HW = H * W
    pp = pool_size * pool_size
    row = C * HW
    win = x[:, :, :pool_size, :pool_size].reshape(N, C, pp)

    region = min(-(-(pool_size * W) // _LANE) * _LANE, row)
    zrows = min(256, N)
    n_chunks = N // zrows
    n_bufs = min(3, n_chunks)

    out2 = pl.pallas_call(
        functools.partial(_extrema_kernel, pool_size=pool_size, width=W,
                          region=region, zrows=zrows, n_chunks=n_chunks,
                          n_bufs=n_bufs),
        out_shape=jax.ShapeDtypeStruct((N, row), x.dtype),
        in_specs=[pl.BlockSpec(memory_space=pl.ANY)],
        out_specs=pl.BlockSpec(memory_space=pl.ANY),
        scratch_shapes=[
            pltpu.VMEM((N, C, pp), x.dtype),
            pltpu.VMEM((n_bufs, zrows, row), x.dtype),
            pltpu.SemaphoreType.DMA((n_bufs,)),
            pltpu.SemaphoreType.DMA,
        ],
        compiler_params=pltpu.CompilerParams(
            vmem_limit_bytes=64 * 1024 * 1024,
        ),
        cost_estimate=pl.CostEstimate(
            flops=10 * N * C * pp + 2 * N * region,
            transcendentals=0,
            bytes_accessed=(N * row + N * C * pp) * x.dtype.itemsize,
        ),
    )(win)
    return out2.reshape(N, C, H, W)


def kernel(x):
    return _extrema_pool_indices_2d(x, 4)


# speedup vs baseline: 1.0207x; 1.0207x over previous
import functools

import jax
import jax.numpy as jnp
from jax import lax
from jax.experimental import pallas as pl
from jax.experimental.pallas import tpu as pltpu

_LANE = 128


def _extrema_kernel(win_hbm, o_hbm, winbuf, bufs, sems, rsem, *,
                    pool_size: int, width: int, region: int, zrows: int,
                    n_chunks: int, n_bufs: int):
    row = o_hbm.shape[1]
    pp = winbuf.shape[2]

    pltpu.make_async_copy(win_hbm, winbuf, rsem).start()
    bufs[0] = jnp.zeros(bufs.shape[1:], bufs.dtype)
    pltpu.make_async_copy(win_hbm, winbuf, rsem).wait()

    dcol = lax.broadcasted_iota(jnp.int32, (1, region), 1)

    for k in range(n_chunks):
        slot = k % n_bufs
        if k >= n_bufs:
            pltpu.make_async_copy(
                bufs.at[slot], o_hbm.at[pl.ds((k - n_bufs) * zrows, zrows), :],
                sems.at[slot]).wait()
        if 0 < k < n_bufs:
            bufs[slot] = jnp.zeros(bufs.shape[1:], bufs.dtype)
        win = winbuf[pl.ds(k * zrows, zrows)]
        awin = jnp.abs(win)
        jidx = jnp.argmax(awin, axis=-1, keepdims=True).astype(jnp.int32)
        wcol2 = lax.broadcasted_iota(jnp.int32, (1, pp), 1)
        hit = wcol2 == jidx[:, 0, :]
        for c in range(1, winbuf.shape[1]):
            hit = hit | (wcol2 == jidx[:, c, :])
        strip = jnp.where(hit, win[:, 0, :], 0.0)
        acc = jnp.zeros((zrows, region), bufs.dtype)
        for j in range(pp):
            acc = jnp.where(dcol == (j // pool_size) * width + j % pool_size,
                            strip[:, j:j + 1], acc)
        bufs[slot, :, :region] = acc
        pltpu.make_async_copy(
            bufs.at[slot], o_hbm.at[pl.ds(k * zrows, zrows), :],
            sems.at[slot]).start()

    for k in range(max(0, n_chunks - n_bufs), n_chunks):
        pltpu.make_async_copy(
            bufs.at[k % n_bufs], o_hbm.at[pl.ds(k * zrows, zrows), :],
            sems.at[k % n_bufs]).wait()


def _extrema_pool_indices_2d(x, pool_size: int):
    N, C, H, W = x.shape
    HW = H * W
    pp = pool_size * pool_size
    row = C * HW
    win = x[:, :, :pool_size, :pool_size].reshape(N, C, pp)

    region = min(-(-(pool_size * W) // _LANE) * _LANE, row)
    zrows = min(256, N)
    n_chunks = N // zrows
    n_bufs = min(2, n_chunks)

    out2 = pl.pallas_call(
        functools.partial(_extrema_kernel, pool_size=pool_size, width=W,
                          region=region, zrows=zrows, n_chunks=n_chunks,
                          n_bufs=n_bufs),
        out_shape=jax.ShapeDtypeStruct((N, row), x.dtype),
        in_specs=[pl.BlockSpec(memory_space=pl.ANY)],
        out_specs=pl.BlockSpec(memory_space=pl.ANY),
        scratch_shapes=[
            pltpu.VMEM((N, C, pp), x.dtype),
            pltpu.VMEM((n_bufs, zrows, row), x.dtype),
            pltpu.SemaphoreType.DMA((n_bufs,)),
            pltpu.SemaphoreType.DMA,
        ],
        compiler_params=pltpu.CompilerParams(
            vmem_limit_bytes=64 * 1024 * 1024,
        ),
        cost_estimate=pl.CostEstimate(
            flops=10 * N * C * pp + 2 * N * region,
            transcendentals=0,
            bytes_accessed=(N * row + N * C * pp) * x.dtype.itemsize,
        ),
    )(win)
    return out2.reshape(N, C, H, W)


def kernel(x):
    return _extrema_pool_indices_2d(x, 4)
